# baseline (speedup 1.0000x reference)
import functools

import jax
import jax.numpy as jnp
from jax import lax
from jax.experimental import pallas as pl
from jax.experimental.pallas import tpu as pltpu
from jax.experimental.pallas import tpu_sc as plsc

N = 10000
HID = 128
E = 160000
NC = 2
NS = 16
NW = NC * NS
CHUNK = 128
NCHUNK0 = 64
NCHUNK1 = 16
NCHUNK = NCHUNK0
EPAD = NS * (NCHUNK0 + NCHUNK1) * CHUNK
RPT = 632
NPAD = NS * RPT

_INV_SQRT2 = 0.7071067811865476

NBUF = 2


def _seg_sum_body(table_hbm, src_hbm, dst_hbm, zeros_hbm, out_hbm,
                  src_v, dst_v, rows_v, acc_sh, gs0, gs1):
    gs = (gs0, gs1)
    c = lax.axis_index("c")
    s = lax.axis_index("s")
    wid = c * NS + s
    pltpu.sync_copy(zeros_hbm.at[pl.ds(s * RPT, RPT)],
                    acc_sh.at[pl.ds(s * RPT, RPT)])
    pltpu.sync_copy(src_hbm.at[wid], src_v)
    pltpu.sync_copy(dst_hbm.at[wid], dst_v)
    plsc.subcore_barrier()

    def _gather(j, b):
        pltpu.async_copy(table_hbm.at[src_v.at[j]], rows_v.at[b], gs[b])

    def _gather_wait(j, b):
        pltpu.make_async_copy(
            table_hbm.at[src_v.at[j]], rows_v.at[b], gs[b]).wait()

    nch = jnp.where(c == 0, NCHUNK0, NCHUNK1)
    for b in range(NBUF):
        _gather(b, b)

    def body(jj, carry):
        for b in range(NBUF):
            j = jj * NBUF + b
            _gather_wait(j, b)
            pltpu.sync_copy(rows_v.at[b], acc_sh.at[dst_v.at[j]], add=True)
            _gather(j + NBUF, b)
        return carry

    lax.fori_loop(0, nch // NBUF - 1, body, 0)
    for b in range(NBUF):
        j = nch - NBUF + b
        _gather_wait(j, b)
        pltpu.sync_copy(rows_v.at[b], acc_sh.at[dst_v.at[j]], add=True)
    plsc.subcore_barrier()
    pltpu.sync_copy(acc_sh.at[pl.ds(s * RPT, RPT)],
                    out_hbm.at[c].at[pl.ds(s * RPT, RPT)])


@functools.cache
def _make_seg_sum():
    mesh = plsc.VectorSubcoreMesh(
        core_axis_name="c", subcore_axis_name="s",
        num_cores=NC, num_subcores=NS)
    return pl.kernel(
        _seg_sum_body,
        out_type=jax.ShapeDtypeStruct((NC, NPAD, HID), jnp.float32),
        mesh=mesh,
        scratch_types=[
            pltpu.VMEM((NCHUNK, CHUNK), jnp.int32),
            pltpu.VMEM((NCHUNK, CHUNK), jnp.int32),
            pltpu.VMEM((NBUF, CHUNK, HID), jnp.float32),
            pltpu.VMEM_SHARED((NPAD, HID), jnp.float32),
        ] + [pltpu.SemaphoreType.DMA] * NBUF,
    )


def _mm_relu_body(p_ref, w_ref, b_ref, o_ref):
    agg = p_ref[0, :N, :] + p_ref[1, :N, :]
    o_ref[...] = jnp.maximum(
        jnp.dot(agg, w_ref[...], preferred_element_type=jnp.float32)
        + b_ref[...], 0.0)


_mm_relu = pl.pallas_call(
    _mm_relu_body,
    out_shape=jax.ShapeDtypeStruct((N, HID), jnp.float32),
)


def _combine_body(p_ref, x_ref, w2_ref, b2_ref, wfc_ref, bfc_ref,
                  eps_ref, gamma_ref, beta_ref, g_ref, hn_ref):
    agg2 = p_ref[0, :N, :] + p_ref[1, :N, :]
    h2 = jnp.dot(agg2, w2_ref[...], preferred_element_type=jnp.float32) + b2_ref[...]
    xv = x_ref[...]
    h1 = jnp.dot(xv, wfc_ref[...], preferred_element_type=jnp.float32) + bfc_ref[...]
    eps = eps_ref[...]
    h = (1.0 - eps) * h1 + eps * h2
    mean = jnp.mean(h, axis=0, keepdims=True)
    cent = h - mean
    var = jnp.mean(cent * cent, axis=0, keepdims=True)
    hn_ref[...] = cent / jnp.sqrt(var + 1e-5) * gamma_ref[...] + beta_ref[...]
    g_ref[:, :HID] = (h * _INV_SQRT2).astype(jnp.bfloat16)
    g_ref[:, HID:] = (xv * _INV_SQRT2).astype(jnp.bfloat16)


_combine = pl.pallas_call(
    _combine_body,
    out_shape=[
        jax.ShapeDtypeStruct((N, 2 * HID), jnp.bfloat16),
        jax.ShapeDtypeStruct((N, HID), jnp.float32),
    ],
)

_BM = 512


def _gram_body(a_ref, b_ref, o_ref):
    o_ref[...] = lax.dot_general(
        a_ref[...], b_ref[...], (((1,), (1,)), ((), ())),
        preferred_element_type=jnp.float32)


_gram = pl.pallas_call(
    _gram_body,
    grid=(pl.cdiv(N, _BM), pl.cdiv(N, _BM)),
    in_specs=[
        pl.BlockSpec((_BM, 2 * HID), lambda i, j: (i, 0)),
        pl.BlockSpec((_BM, 2 * HID), lambda i, j: (j, 0)),
    ],
    out_specs=pl.BlockSpec((_BM, _BM), lambda i, j: (i, j)),
    out_shape=jax.ShapeDtypeStruct((N, N), jnp.float32),
    compiler_params=pltpu.CompilerParams(
        dimension_semantics=("parallel", "parallel")),
)


def kernel(x, edge_index, adj, W_fc, b_fc, W1, b1, W2, b2, epsilon, gamma, beta):
    src = edge_index[0]
    dst = edge_index[1]
    pad_src = jnp.zeros((EPAD - E,), dtype=jnp.int32)
    pad_dst = jnp.full((EPAD - E,), N, dtype=jnp.int32)
    e0 = NS * NCHUNK0 * CHUNK

    def _layout(idx, pad):
        flat = jnp.concatenate([idx, pad])
        p0 = flat[:e0].reshape(NS, NCHUNK0, CHUNK)
        p1 = flat[e0:].reshape(NS, NCHUNK1, CHUNK)
        p1 = jnp.pad(p1, ((0, 0), (0, NCHUNK0 - NCHUNK1), (0, 0)))
        return jnp.concatenate([p0, p1], axis=0)

    src_r = _layout(src, pad_src)
    dst_r = _layout(dst, pad_dst)
    zeros_nh = jnp.zeros((NPAD, HID), dtype=jnp.float32)

    seg_sum = _make_seg_sum()
    p1 = seg_sum(x, src_r, dst_r, zeros_nh)
    h2a = _mm_relu(p1, W1, b1.reshape(1, HID))
    p2 = seg_sum(h2a, src_r, dst_r, zeros_nh)
    g, hn = _combine(p2, x, W2, b2.reshape(1, HID), W_fc, b_fc.reshape(1, HID),
                     epsilon.reshape(N, 1), gamma.reshape(1, HID),
                     beta.reshape(1, HID))
    ret = _gram(g, g)
    return (ret, hn)


# speedup vs baseline: 1.0357x; 1.0357x over previous
import functools

import jax
import jax.numpy as jnp
from jax import lax
from jax.experimental import pallas as pl
from jax.experimental.pallas import tpu as pltpu
from jax.experimental.pallas import tpu_sc as plsc

N = 10000
HID = 128
E = 160000
NC = 2
NS = 16
NW = NC * NS
CHUNK = 128
NCHUNK0 = 60
NCHUNK1 = 20
NCHUNK = NCHUNK0
EPAD = NS * (NCHUNK0 + NCHUNK1) * CHUNK
RPT = 632
NPAD = NS * RPT

_INV_SQRT2 = 0.7071067811865476

NBUF = 2


def _seg_sum_body(table_hbm, src_hbm, dst_hbm, zeros_hbm, out_hbm,
                  src_v, dst_v, rows_v, acc_sh, gs0, gs1):
    gs = (gs0, gs1)
    c = lax.axis_index("c")
    s = lax.axis_index("s")
    wid = c * NS + s
    pltpu.sync_copy(zeros_hbm.at[pl.ds(s * RPT, RPT)],
                    acc_sh.at[pl.ds(s * RPT, RPT)])
    pltpu.sync_copy(src_hbm.at[wid], src_v)
    pltpu.sync_copy(dst_hbm.at[wid], dst_v)
    plsc.subcore_barrier()

    def _gather(j, b):
        pltpu.async_copy(table_hbm.at[src_v.at[j]], rows_v.at[b], gs[b])

    def _gather_wait(j, b):
        pltpu.make_async_copy(
            table_hbm.at[src_v.at[j]], rows_v.at[b], gs[b]).wait()

    nch = jnp.where(c == 0, NCHUNK0, NCHUNK1)
    for b in range(NBUF):
        _gather(b, b)

    def body(jj, carry):
        for b in range(NBUF):
            j = jj * NBUF + b
            _gather_wait(j, b)
            pltpu.sync_copy(rows_v.at[b], acc_sh.at[dst_v.at[j]], add=True)
            _gather(j + NBUF, b)
        return carry

    lax.fori_loop(0, nch // NBUF - 1, body, 0)
    for b in range(NBUF):
        j = nch - NBUF + b
        _gather_wait(j, b)
        pltpu.sync_copy(rows_v.at[b], acc_sh.at[dst_v.at[j]], add=True)
    plsc.subcore_barrier()
    pltpu.sync_copy(acc_sh.at[pl.ds(s * RPT, RPT)],
                    out_hbm.at[c].at[pl.ds(s * RPT, RPT)])


@functools.cache
def _make_seg_sum():
    mesh = plsc.VectorSubcoreMesh(
        core_axis_name="c", subcore_axis_name="s",
        num_cores=NC, num_subcores=NS)
    return pl.kernel(
        _seg_sum_body,
        out_type=jax.ShapeDtypeStruct((NC, NPAD, HID), jnp.float32),
        mesh=mesh,
        scratch_types=[
            pltpu.VMEM((NCHUNK, CHUNK), jnp.int32),
            pltpu.VMEM((NCHUNK, CHUNK), jnp.int32),
            pltpu.VMEM((NBUF, CHUNK, HID), jnp.float32),
            pltpu.VMEM_SHARED((NPAD, HID), jnp.float32),
        ] + [pltpu.SemaphoreType.DMA] * NBUF,
    )


def _mm_relu_body(p_ref, w_ref, b_ref, o_ref):
    agg = p_ref[0, :N, :] + p_ref[1, :N, :]
    o_ref[...] = jnp.maximum(
        jnp.dot(agg, w_ref[...], preferred_element_type=jnp.float32)
        + b_ref[...], 0.0)


_mm_relu = pl.pallas_call(
    _mm_relu_body,
    out_shape=jax.ShapeDtypeStruct((N, HID), jnp.float32),
)


def _combine_body(p_ref, x_ref, w2_ref, b2_ref, wfc_ref, bfc_ref,
                  eps_ref, gamma_ref, beta_ref, g_ref, hn_ref):
    agg2 = p_ref[0, :N, :] + p_ref[1, :N, :]
    h2 = jnp.dot(agg2, w2_ref[...], preferred_element_type=jnp.float32) + b2_ref[...]
    xv = x_ref[...]
    h1 = jnp.dot(xv, wfc_ref[...], preferred_element_type=jnp.float32) + bfc_ref[...]
    eps = eps_ref[...]
    h = (1.0 - eps) * h1 + eps * h2
    mean = jnp.mean(h, axis=0, keepdims=True)
    cent = h - mean
    var = jnp.mean(cent * cent, axis=0, keepdims=True)
    hn_ref[...] = cent / jnp.sqrt(var + 1e-5) * gamma_ref[...] + beta_ref[...]
    g_ref[:, :HID] = (h * _INV_SQRT2).astype(jnp.bfloat16)
    g_ref[:, HID:] = (xv * _INV_SQRT2).astype(jnp.bfloat16)


_combine = pl.pallas_call(
    _combine_body,
    out_shape=[
        jax.ShapeDtypeStruct((N, 2 * HID), jnp.bfloat16),
        jax.ShapeDtypeStruct((N, HID), jnp.float32),
    ],
)

_BM = 512


def _gram_body(a_ref, b_ref, o_ref):
    o_ref[...] = lax.dot_general(
        a_ref[...], b_ref[...], (((1,), (1,)), ((), ())),
        preferred_element_type=jnp.float32)


_gram = pl.pallas_call(
    _gram_body,
    grid=(pl.cdiv(N, _BM), pl.cdiv(N, _BM)),
    in_specs=[
        pl.BlockSpec((_BM, 2 * HID), lambda i, j: (i, 0)),
        pl.BlockSpec((_BM, 2 * HID), lambda i, j: (j, 0)),
    ],
    out_specs=pl.BlockSpec((_BM, _BM), lambda i, j: (i, j)),
    out_shape=jax.ShapeDtypeStruct((N, N), jnp.float32),
    compiler_params=pltpu.CompilerParams(
        dimension_semantics=("parallel", "parallel")),
)


def kernel(x, edge_index, adj, W_fc, b_fc, W1, b1, W2, b2, epsilon, gamma, beta):
    src = edge_index[0]
    dst = edge_index[1]
    pad_src = jnp.zeros((EPAD - E,), dtype=jnp.int32)
    pad_dst = jnp.full((EPAD - E,), N, dtype=jnp.int32)
    e0 = NS * NCHUNK0 * CHUNK

    def _layout(idx, pad):
        flat = jnp.concatenate([idx, pad])
        p0 = flat[:e0].reshape(NS, NCHUNK0, CHUNK)
        p1 = flat[e0:].reshape(NS, NCHUNK1, CHUNK)
        p1 = jnp.pad(p1, ((0, 0), (0, NCHUNK0 - NCHUNK1), (0, 0)))
        return jnp.concatenate([p0, p1], axis=0)

    src_r = _layout(src, pad_src)
    dst_r = _layout(dst, pad_dst)
    zeros_nh = jnp.zeros((NPAD, HID), dtype=jnp.float32)

    seg_sum = _make_seg_sum()
    p1 = seg_sum(x, src_r, dst_r, zeros_nh)
    h2a = _mm_relu(p1, W1, b1.reshape(1, HID))
    p2 = seg_sum(h2a, src_r, dst_r, zeros_nh)
    g, hn = _combine(p2, x, W2, b2.reshape(1, HID), W_fc, b_fc.reshape(1, HID),
                     epsilon.reshape(N, 1), gamma.reshape(1, HID),
                     beta.reshape(1, HID))
    ret = _gram(g, g)
    return (ret, hn)


# speedup vs baseline: 1.4098x; 1.3612x over previous
import functools

import jax
import jax.numpy as jnp
from jax import lax
from jax.experimental import pallas as pl
from jax.experimental.pallas import tpu as pltpu
from jax.experimental.pallas import tpu_sc as plsc

N = 10000
HID = 128
E = 160000
NC = 2
NS = 16
NW = NC * NS
CHUNK = 128
NCHUNK0 = 60
NCHUNK1 = 20
NCHUNK = NCHUNK0
EPAD = NS * (NCHUNK0 + NCHUNK1) * CHUNK
RPT = 632
NPAD = NS * RPT

_INV_SQRT2 = 0.7071067811865476

NBUF = 2


def _seg_sum_body(table_hbm, src_hbm, dst_hbm, zeros_hbm, out_hbm,
                  src_v, dst_v, rows_v, acc_sh, gs0, gs1):
    gs = (gs0, gs1)
    c = lax.axis_index("c")
    s = lax.axis_index("s")
    wid = c * NS + s
    pltpu.sync_copy(zeros_hbm.at[pl.ds(s * RPT, RPT)],
                    acc_sh.at[pl.ds(s * RPT, RPT)])
    pltpu.sync_copy(src_hbm.at[wid], src_v)
    pltpu.sync_copy(dst_hbm.at[wid], dst_v)
    plsc.subcore_barrier()

    def _gather(j, b):
        pltpu.async_copy(table_hbm.at[src_v.at[j]], rows_v.at[b], gs[b])

    def _gather_wait(j, b):
        pltpu.make_async_copy(
            table_hbm.at[src_v.at[j]], rows_v.at[b], gs[b]).wait()

    nch = jnp.where(c == 0, NCHUNK0, NCHUNK1)
    for b in range(NBUF):
        _gather(b, b)

    def body(jj, carry):
        for b in range(NBUF):
            j = jj * NBUF + b
            _gather_wait(j, b)
            pltpu.sync_copy(rows_v.at[b], acc_sh.at[dst_v.at[j]], add=True)
            _gather(j + NBUF, b)
        return carry

    lax.fori_loop(0, nch // NBUF - 1, body, 0)
    for b in range(NBUF):
        j = nch - NBUF + b
        _gather_wait(j, b)
        pltpu.sync_copy(rows_v.at[b], acc_sh.at[dst_v.at[j]], add=True)
    plsc.subcore_barrier()
    pltpu.sync_copy(acc_sh.at[pl.ds(s * RPT, RPT)],
                    out_hbm.at[c].at[pl.ds(s * RPT, RPT)])


@functools.cache
def _make_seg_sum():
    mesh = plsc.VectorSubcoreMesh(
        core_axis_name="c", subcore_axis_name="s",
        num_cores=NC, num_subcores=NS)
    return pl.kernel(
        _seg_sum_body,
        out_type=jax.ShapeDtypeStruct((NC, NPAD, HID), jnp.float32),
        mesh=mesh,
        scratch_types=[
            pltpu.VMEM((NCHUNK, CHUNK), jnp.int32),
            pltpu.VMEM((NCHUNK, CHUNK), jnp.int32),
            pltpu.VMEM((NBUF, CHUNK, HID), jnp.float32),
            pltpu.VMEM_SHARED((NPAD, HID), jnp.float32),
        ] + [pltpu.SemaphoreType.DMA] * NBUF,
    )


def _mm_relu_body(p_ref, w_ref, b_ref, o_ref):
    agg = p_ref[0, :N, :] + p_ref[1, :N, :]
    o_ref[...] = jnp.maximum(
        jnp.dot(agg, w_ref[...], preferred_element_type=jnp.float32)
        + b_ref[...], 0.0)


_mm_relu = pl.pallas_call(
    _mm_relu_body,
    out_shape=jax.ShapeDtypeStruct((N, HID), jnp.float32),
)


def _combine_body(p_ref, x_ref, w2_ref, b2_ref, wfc_ref, bfc_ref,
                  eps_ref, gamma_ref, beta_ref, g_ref, hn_ref):
    agg2 = p_ref[0, :N, :] + p_ref[1, :N, :]
    h2 = jnp.dot(agg2, w2_ref[...], preferred_element_type=jnp.float32) + b2_ref[...]
    xv = x_ref[...]
    h1 = jnp.dot(xv, wfc_ref[...], preferred_element_type=jnp.float32) + bfc_ref[...]
    eps = eps_ref[...]
    h = (1.0 - eps) * h1 + eps * h2
    mean = jnp.mean(h, axis=0, keepdims=True)
    cent = h - mean
    var = jnp.mean(cent * cent, axis=0, keepdims=True)
    hn_ref[...] = cent / jnp.sqrt(var + 1e-5) * gamma_ref[...] + beta_ref[...]
    g_ref[:, :HID] = (h * _INV_SQRT2).astype(jnp.bfloat16)
    g_ref[:, HID:] = (xv * _INV_SQRT2).astype(jnp.bfloat16)


_combine = pl.pallas_call(
    _combine_body,
    out_shape=[
        jax.ShapeDtypeStruct((N, 2 * HID), jnp.bfloat16),
        jax.ShapeDtypeStruct((N, HID), jnp.float32),
    ],
)

_BM = 1024


def _gram_body(a_ref, b_ref, o_ref):
    o_ref[...] = lax.dot_general(
        a_ref[...], b_ref[...], (((1,), (1,)), ((), ())),
        preferred_element_type=jnp.float32)


_gram = pl.pallas_call(
    _gram_body,
    grid=(pl.cdiv(N, _BM), pl.cdiv(N, _BM)),
    in_specs=[
        pl.BlockSpec((_BM, 2 * HID), lambda i, j: (i, 0)),
        pl.BlockSpec((_BM, 2 * HID), lambda i, j: (j, 0)),
    ],
    out_specs=pl.BlockSpec((_BM, _BM), lambda i, j: (i, j)),
    out_shape=jax.ShapeDtypeStruct((N, N), jnp.float32),
    compiler_params=pltpu.CompilerParams(
        dimension_semantics=("parallel", "parallel")),
)


def kernel(x, edge_index, adj, W_fc, b_fc, W1, b1, W2, b2, epsilon, gamma, beta):
    src = edge_index[0]
    dst = edge_index[1]
    pad_src = jnp.full((EPAD - E,), N, dtype=jnp.int32)
    pad_dst = jnp.zeros((EPAD - E,), dtype=jnp.int32)
    e0 = NS * NCHUNK0 * CHUNK

    def _layout(idx, pad):
        flat = jnp.concatenate([idx, pad])
        p0 = flat[:e0].reshape(NS, NCHUNK0, CHUNK)
        p1 = flat[e0:].reshape(NS, NCHUNK1, CHUNK)
        p1 = jnp.pad(p1, ((0, 0), (0, NCHUNK0 - NCHUNK1), (0, 0)))
        return jnp.concatenate([p0, p1], axis=0)

    src_r = _layout(src, pad_src)
    dst_r = _layout(dst, pad_dst)
    zrow = jnp.zeros((1, HID), dtype=jnp.float32)
    zeros_nh = jnp.zeros((NPAD, HID), dtype=jnp.float32)

    seg_sum = _make_seg_sum()
    table1 = jnp.concatenate([x, zrow], axis=0)
    p1 = seg_sum(table1, src_r, dst_r, zeros_nh)
    h2a = _mm_relu(p1, W1, b1.reshape(1, HID))
    table2 = jnp.concatenate([h2a, zrow], axis=0)
    p2 = seg_sum(table2, src_r, dst_r, zeros_nh)
    g, hn = _combine(p2, x, W2, b2.reshape(1, HID), W_fc, b_fc.reshape(1, HID),
                     epsilon.reshape(N, 1), gamma.reshape(1, HID),
                     beta.reshape(1, HID))
    ret = _gram(g, g)
    return (ret, hn)


# speedup vs baseline: 1.4873x; 1.0550x over previous
import functools

import jax
import jax.numpy as jnp
from jax import lax
from jax.experimental import pallas as pl
from jax.experimental.pallas import tpu as pltpu
from jax.experimental.pallas import tpu_sc as plsc

N = 10000
HID = 128
E = 160000
NC = 2
NS = 16
NW = NC * NS
CHUNK = 128
NCHUNK0 = 60
NCHUNK1 = 20
NCHUNK = NCHUNK0
EPAD = NS * (NCHUNK0 + NCHUNK1) * CHUNK
RPT = 632
NPAD = NS * RPT

_INV_SQRT2 = 0.7071067811865476

NBUF = 2


def _seg_sum_body(table_hbm, src_hbm, dst_hbm, zeros_hbm, out_hbm,
                  src_v, dst_v, rows_v, acc_sh, gs0, gs1):
    gs = (gs0, gs1)
    c = lax.axis_index("c")
    s = lax.axis_index("s")
    wid = c * NS + s
    pltpu.sync_copy(zeros_hbm.at[pl.ds(s * RPT, RPT)],
                    acc_sh.at[pl.ds(s * RPT, RPT)])
    pltpu.sync_copy(src_hbm.at[wid], src_v)
    pltpu.sync_copy(dst_hbm.at[wid], dst_v)
    plsc.subcore_barrier()

    def _gather(j, b):
        pltpu.async_copy(table_hbm.at[src_v.at[j]], rows_v.at[b], gs[b])

    def _gather_wait(j, b):
        pltpu.make_async_copy(
            table_hbm.at[src_v.at[j]], rows_v.at[b], gs[b]).wait()

    nch = jnp.where(c == 0, NCHUNK0, NCHUNK1)
    for b in range(NBUF):
        _gather(b, b)

    def body(jj, carry):
        for b in range(NBUF):
            j = jj * NBUF + b
            _gather_wait(j, b)
            pltpu.sync_copy(rows_v.at[b], acc_sh.at[dst_v.at[j]], add=True)
            _gather(j + NBUF, b)
        return carry

    lax.fori_loop(0, nch // NBUF - 1, body, 0)
    for b in range(NBUF):
        j = nch - NBUF + b
        _gather_wait(j, b)
        pltpu.sync_copy(rows_v.at[b], acc_sh.at[dst_v.at[j]], add=True)
    plsc.subcore_barrier()
    pltpu.sync_copy(acc_sh.at[pl.ds(s * RPT, RPT)],
                    out_hbm.at[c].at[pl.ds(s * RPT, RPT)])


@functools.cache
def _make_seg_sum():
    mesh = plsc.VectorSubcoreMesh(
        core_axis_name="c", subcore_axis_name="s",
        num_cores=NC, num_subcores=NS)
    return pl.kernel(
        _seg_sum_body,
        out_type=jax.ShapeDtypeStruct((NC, NPAD, HID), jnp.float32),
        mesh=mesh,
        scratch_types=[
            pltpu.VMEM((NCHUNK, CHUNK), jnp.int32),
            pltpu.VMEM((NCHUNK, CHUNK), jnp.int32),
            pltpu.VMEM((NBUF, CHUNK, HID), jnp.float32),
            pltpu.VMEM_SHARED((NPAD, HID), jnp.float32),
        ] + [pltpu.SemaphoreType.DMA] * NBUF,
    )


def _mm_relu_body(p_ref, w_ref, b_ref, o_ref):
    agg = p_ref[0, :N, :] + p_ref[1, :N, :]
    o_ref[...] = jnp.maximum(
        jnp.dot(agg, w_ref[...], preferred_element_type=jnp.float32)
        + b_ref[...], 0.0)


_mm_relu = pl.pallas_call(
    _mm_relu_body,
    out_shape=jax.ShapeDtypeStruct((N, HID), jnp.float32),
)


def _combine_body(p_ref, x_ref, w2_ref, b2_ref, wfc_ref, bfc_ref,
                  eps_ref, gamma_ref, beta_ref, g_ref, hn_ref):
    agg2 = p_ref[0, :N, :] + p_ref[1, :N, :]
    h2 = jnp.dot(agg2, w2_ref[...], preferred_element_type=jnp.float32) + b2_ref[...]
    xv = x_ref[...]
    h1 = jnp.dot(xv, wfc_ref[...], preferred_element_type=jnp.float32) + bfc_ref[...]
    eps = eps_ref[...]
    h = (1.0 - eps) * h1 + eps * h2
    mean = jnp.mean(h, axis=0, keepdims=True)
    cent = h - mean
    var = jnp.mean(cent * cent, axis=0, keepdims=True)
    hn_ref[...] = cent / jnp.sqrt(var + 1e-5) * gamma_ref[...] + beta_ref[...]
    g_ref[:, :HID] = (h * _INV_SQRT2).astype(jnp.bfloat16)
    g_ref[:, HID:] = (xv * _INV_SQRT2).astype(jnp.bfloat16)


_combine = pl.pallas_call(
    _combine_body,
    out_shape=[
        jax.ShapeDtypeStruct((N, 2 * HID), jnp.bfloat16),
        jax.ShapeDtypeStruct((N, HID), jnp.float32),
    ],
)

_BM = 2048


def _gram_body(a_ref, b_ref, o_ref):
    o_ref[...] = lax.dot_general(
        a_ref[...], b_ref[...], (((1,), (1,)), ((), ())),
        preferred_element_type=jnp.float32)


_gram = pl.pallas_call(
    _gram_body,
    grid=(pl.cdiv(N, _BM), pl.cdiv(N, _BM)),
    in_specs=[
        pl.BlockSpec((_BM, 2 * HID), lambda i, j: (i, 0)),
        pl.BlockSpec((_BM, 2 * HID), lambda i, j: (j, 0)),
    ],
    out_specs=pl.BlockSpec((_BM, _BM), lambda i, j: (i, j)),
    out_shape=jax.ShapeDtypeStruct((N, N), jnp.float32),
    compiler_params=pltpu.CompilerParams(
        dimension_semantics=("parallel", "parallel")),
)


def kernel(x, edge_index, adj, W_fc, b_fc, W1, b1, W2, b2, epsilon, gamma, beta):
    src = edge_index[0]
    dst = edge_index[1]
    pad_src = jnp.full((EPAD - E,), N, dtype=jnp.int32)
    pad_dst = jnp.zeros((EPAD - E,), dtype=jnp.int32)
    e0 = NS * NCHUNK0 * CHUNK

    def _layout(idx, pad):
        flat = jnp.concatenate([idx, pad])
        p0 = flat[:e0].reshape(NS, NCHUNK0, CHUNK)
        p1 = flat[e0:].reshape(NS, NCHUNK1, CHUNK)
        p1 = jnp.pad(p1, ((0, 0), (0, NCHUNK0 - NCHUNK1), (0, 0)))
        return jnp.concatenate([p0, p1], axis=0)

    src_r = _layout(src, pad_src)
    dst_r = _layout(dst, pad_dst)
    zrow = jnp.zeros((1, HID), dtype=jnp.float32)
    zeros_nh = jnp.zeros((NPAD, HID), dtype=jnp.float32)

    seg_sum = _make_seg_sum()
    table1 = jnp.concatenate([x, zrow], axis=0)
    p1 = seg_sum(table1, src_r, dst_r, zeros_nh)
    h2a = _mm_relu(p1, W1, b1.reshape(1, HID))
    table2 = jnp.concatenate([h2a, zrow], axis=0)
    p2 = seg_sum(table2, src_r, dst_r, zeros_nh)
    g, hn = _combine(p2, x, W2, b2.reshape(1, HID), W_fc, b_fc.reshape(1, HID),
                     epsilon.reshape(N, 1), gamma.reshape(1, HID),
                     beta.reshape(1, HID))
    ret = _gram(g, g)
    return (ret, hn)


# speedup vs baseline: 1.4918x; 1.0030x over previous
import functools

import jax
import jax.numpy as jnp
from jax import lax
from jax.experimental import pallas as pl
from jax.experimental.pallas import tpu as pltpu
from jax.experimental.pallas import tpu_sc as plsc

N = 10000
HID = 128
E = 160000
NC = 2
NS = 16
NW = NC * NS
CHUNK = 128
NCHUNK0 = 60
NCHUNK1 = 20
NCHUNK = NCHUNK0
EPAD = NS * (NCHUNK0 + NCHUNK1) * CHUNK
RPT = 632
NPAD = NS * RPT

_INV_SQRT2 = 0.7071067811865476

NBUF = 2


def _seg_sum_body(table_hbm, src_hbm, dst_hbm, zeros_hbm, out_hbm,
                  src_v, dst_v, rows_v, acc_sh, gs0, gs1):
    gs = (gs0, gs1)
    c = lax.axis_index("c")
    s = lax.axis_index("s")
    wid = c * NS + s
    pltpu.sync_copy(zeros_hbm.at[pl.ds(s * RPT, RPT)],
                    acc_sh.at[pl.ds(s * RPT, RPT)])
    pltpu.sync_copy(src_hbm.at[wid], src_v)
    pltpu.sync_copy(dst_hbm.at[wid], dst_v)
    plsc.subcore_barrier()

    def _gather(j, b):
        pltpu.async_copy(table_hbm.at[src_v.at[j]], rows_v.at[b], gs[b])

    def _gather_wait(j, b):
        pltpu.make_async_copy(
            table_hbm.at[src_v.at[j]], rows_v.at[b], gs[b]).wait()

    nch = jnp.where(c == 0, NCHUNK0, NCHUNK1)
    for b in range(NBUF):
        _gather(b, b)

    def body(jj, carry):
        for b in range(NBUF):
            j = jj * NBUF + b
            _gather_wait(j, b)
            pltpu.sync_copy(rows_v.at[b], acc_sh.at[dst_v.at[j]], add=True)
            _gather(j + NBUF, b)
        return carry

    lax.fori_loop(0, nch // NBUF - 1, body, 0)
    for b in range(NBUF):
        j = nch - NBUF + b
        _gather_wait(j, b)
        pltpu.sync_copy(rows_v.at[b], acc_sh.at[dst_v.at[j]], add=True)
    plsc.subcore_barrier()
    pltpu.sync_copy(acc_sh.at[pl.ds(s * RPT, RPT)],
                    out_hbm.at[c].at[pl.ds(s * RPT, RPT)])


@functools.cache
def _make_seg_sum():
    mesh = plsc.VectorSubcoreMesh(
        core_axis_name="c", subcore_axis_name="s",
        num_cores=NC, num_subcores=NS)
    return pl.kernel(
        _seg_sum_body,
        out_type=jax.ShapeDtypeStruct((NC, NPAD, HID), jnp.float32),
        mesh=mesh,
        scratch_types=[
            pltpu.VMEM((NCHUNK, CHUNK), jnp.int32),
            pltpu.VMEM((NCHUNK, CHUNK), jnp.int32),
            pltpu.VMEM((NBUF, CHUNK, HID), jnp.float32),
            pltpu.VMEM_SHARED((NPAD, HID), jnp.float32),
        ] + [pltpu.SemaphoreType.DMA] * NBUF,
    )


def _mm_relu_body(p_ref, w_ref, b_ref, o_ref):
    agg = p_ref[0, :N, :] + p_ref[1, :N, :]
    o_ref[...] = jnp.maximum(
        jnp.dot(agg, w_ref[...], preferred_element_type=jnp.float32)
        + b_ref[...], 0.0)


_mm_relu = pl.pallas_call(
    _mm_relu_body,
    out_shape=jax.ShapeDtypeStruct((N, HID), jnp.float32),
)


def _combine_body(p_ref, x_ref, w2_ref, b2_ref, wfc_ref, bfc_ref,
                  eps_ref, gamma_ref, beta_ref, g_ref, hn_ref):
    agg2 = p_ref[0, :N, :] + p_ref[1, :N, :]
    h2 = jnp.dot(agg2, w2_ref[...], preferred_element_type=jnp.float32) + b2_ref[...]
    xv = x_ref[...]
    h1 = jnp.dot(xv, wfc_ref[...], preferred_element_type=jnp.float32) + bfc_ref[...]
    eps = eps_ref[...]
    h = (1.0 - eps) * h1 + eps * h2
    mean = jnp.mean(h, axis=0, keepdims=True)
    cent = h - mean
    var = jnp.mean(cent * cent, axis=0, keepdims=True)
    hn_ref[...] = cent / jnp.sqrt(var + 1e-5) * gamma_ref[...] + beta_ref[...]
    g_ref[:, :HID] = (h * _INV_SQRT2).astype(jnp.bfloat16)
    g_ref[:, HID:] = (xv * _INV_SQRT2).astype(jnp.bfloat16)


_combine = pl.pallas_call(
    _combine_body,
    out_shape=[
        jax.ShapeDtypeStruct((N, 2 * HID), jnp.bfloat16),
        jax.ShapeDtypeStruct((N, HID), jnp.float32),
    ],
)

_BM = 2560


def _gram_body(a_ref, b_ref, o_ref):
    o_ref[...] = lax.dot_general(
        a_ref[...], b_ref[...], (((1,), (1,)), ((), ())),
        preferred_element_type=jnp.float32)


_gram = pl.pallas_call(
    _gram_body,
    grid=(pl.cdiv(N, _BM), pl.cdiv(N, _BM)),
    in_specs=[
        pl.BlockSpec((_BM, 2 * HID), lambda i, j: (i, 0)),
        pl.BlockSpec((_BM, 2 * HID), lambda i, j: (j, 0)),
    ],
    out_specs=pl.BlockSpec((_BM, _BM), lambda i, j: (i, j)),
    out_shape=jax.ShapeDtypeStruct((N, N), jnp.float32),
    compiler_params=pltpu.CompilerParams(
        dimension_semantics=("parallel", "parallel")),
)


def kernel(x, edge_index, adj, W_fc, b_fc, W1, b1, W2, b2, epsilon, gamma, beta):
    src = edge_index[0]
    dst = edge_index[1]
    pad_src = jnp.full((EPAD - E,), N, dtype=jnp.int32)
    pad_dst = jnp.zeros((EPAD - E,), dtype=jnp.int32)
    e0 = NS * NCHUNK0 * CHUNK

    def _layout(idx, pad):
        flat = jnp.concatenate([idx, pad])
        p0 = flat[:e0].reshape(NS, NCHUNK0, CHUNK)
        p1 = flat[e0:].reshape(NS, NCHUNK1, CHUNK)
        p1 = jnp.pad(p1, ((0, 0), (0, NCHUNK0 - NCHUNK1), (0, 0)))
        return jnp.concatenate([p0, p1], axis=0)

    src_r = _layout(src, pad_src)
    dst_r = _layout(dst, pad_dst)
    zrow = jnp.zeros((1, HID), dtype=jnp.float32)
    zeros_nh = jnp.zeros((NPAD, HID), dtype=jnp.float32)

    seg_sum = _make_seg_sum()
    table1 = jnp.concatenate([x, zrow], axis=0)
    p1 = seg_sum(table1, src_r, dst_r, zeros_nh)
    h2a = _mm_relu(p1, W1, b1.reshape(1, HID))
    table2 = jnp.concatenate([h2a, zrow], axis=0)
    p2 = seg_sum(table2, src_r, dst_r, zeros_nh)
    g, hn = _combine(p2, x, W2, b2.reshape(1, HID), W_fc, b_fc.reshape(1, HID),
                     epsilon.reshape(N, 1), gamma.reshape(1, HID),
                     beta.reshape(1, HID))
    ret = _gram(g, g)
    return (ret, hn)


# speedup vs baseline: 1.4949x; 1.0021x over previous
import functools

import jax
import jax.numpy as jnp
from jax import lax
from jax.experimental import pallas as pl
from jax.experimental.pallas import tpu as pltpu
from jax.experimental.pallas import tpu_sc as plsc

N = 10000
HID = 128
E = 160000
NC = 2
NS = 16
NW = NC * NS
CHUNK = 128
NCHUNK0 = 56
NCHUNK1 = 24
NCHUNK = NCHUNK0
EPAD = NS * (NCHUNK0 + NCHUNK1) * CHUNK
RPT = 632
NPAD = NS * RPT

_INV_SQRT2 = 0.7071067811865476

NBUF = 2


def _seg_sum_body(table_hbm, src_hbm, dst_hbm, zeros_hbm, out_hbm,
                  src_v, dst_v, rows_v, acc_sh, gs0, gs1):
    gs = (gs0, gs1)
    c = lax.axis_index("c")
    s = lax.axis_index("s")
    wid = c * NS + s
    pltpu.sync_copy(zeros_hbm.at[pl.ds(s * RPT, RPT)],
                    acc_sh.at[pl.ds(s * RPT, RPT)])
    pltpu.sync_copy(src_hbm.at[wid], src_v)
    pltpu.sync_copy(dst_hbm.at[wid], dst_v)
    plsc.subcore_barrier()

    def _gather(j, b):
        pltpu.async_copy(table_hbm.at[src_v.at[j]], rows_v.at[b], gs[b])

    def _gather_wait(j, b):
        pltpu.make_async_copy(
            table_hbm.at[src_v.at[j]], rows_v.at[b], gs[b]).wait()

    nch = jnp.where(c == 0, NCHUNK0, NCHUNK1)
    for b in range(NBUF):
        _gather(b, b)

    def body(jj, carry):
        for b in range(NBUF):
            j = jj * NBUF + b
            _gather_wait(j, b)
            pltpu.sync_copy(rows_v.at[b], acc_sh.at[dst_v.at[j]], add=True)
            _gather(j + NBUF, b)
        return carry

    lax.fori_loop(0, nch // NBUF - 1, body, 0)
    for b in range(NBUF):
        j = nch - NBUF + b
        _gather_wait(j, b)
        pltpu.sync_copy(rows_v.at[b], acc_sh.at[dst_v.at[j]], add=True)
    plsc.subcore_barrier()
    pltpu.sync_copy(acc_sh.at[pl.ds(s * RPT, RPT)],
                    out_hbm.at[c].at[pl.ds(s * RPT, RPT)])


@functools.cache
def _make_seg_sum():
    mesh = plsc.VectorSubcoreMesh(
        core_axis_name="c", subcore_axis_name="s",
        num_cores=NC, num_subcores=NS)
    return pl.kernel(
        _seg_sum_body,
        out_type=jax.ShapeDtypeStruct((NC, NPAD, HID), jnp.float32),
        mesh=mesh,
        scratch_types=[
            pltpu.VMEM((NCHUNK, CHUNK), jnp.int32),
            pltpu.VMEM((NCHUNK, CHUNK), jnp.int32),
            pltpu.VMEM((NBUF, CHUNK, HID), jnp.float32),
            pltpu.VMEM_SHARED((NPAD, HID), jnp.float32),
        ] + [pltpu.SemaphoreType.DMA] * NBUF,
    )


def _mm_relu_body(p_ref, w_ref, b_ref, o_ref):
    agg = p_ref[0, :N, :] + p_ref[1, :N, :]
    o_ref[...] = jnp.maximum(
        jnp.dot(agg, w_ref[...], preferred_element_type=jnp.float32)
        + b_ref[...], 0.0)


_mm_relu = pl.pallas_call(
    _mm_relu_body,
    out_shape=jax.ShapeDtypeStruct((N, HID), jnp.float32),
)


def _combine_body(p_ref, x_ref, w2_ref, b2_ref, wfc_ref, bfc_ref,
                  eps_ref, gamma_ref, beta_ref, g_ref, hn_ref):
    agg2 = p_ref[0, :N, :] + p_ref[1, :N, :]
    h2 = jnp.dot(agg2, w2_ref[...], preferred_element_type=jnp.float32) + b2_ref[...]
    xv = x_ref[...]
    h1 = jnp.dot(xv, wfc_ref[...], preferred_element_type=jnp.float32) + bfc_ref[...]
    eps = eps_ref[...]
    h = (1.0 - eps) * h1 + eps * h2
    mean = jnp.mean(h, axis=0, keepdims=True)
    cent = h - mean
    var = jnp.mean(cent * cent, axis=0, keepdims=True)
    hn_ref[...] = cent / jnp.sqrt(var + 1e-5) * gamma_ref[...] + beta_ref[...]
    g_ref[:, :HID] = (h * _INV_SQRT2).astype(jnp.bfloat16)
    g_ref[:, HID:] = (xv * _INV_SQRT2).astype(jnp.bfloat16)


_combine = pl.pallas_call(
    _combine_body,
    out_shape=[
        jax.ShapeDtypeStruct((N, 2 * HID), jnp.bfloat16),
        jax.ShapeDtypeStruct((N, HID), jnp.float32),
    ],
)

_BM = 2560


def _gram_body(a_ref, b_ref, o_ref):
    o_ref[...] = lax.dot_general(
        a_ref[...], b_ref[...], (((1,), (1,)), ((), ())),
        preferred_element_type=jnp.float32)


_gram = pl.pallas_call(
    _gram_body,
    grid=(pl.cdiv(N, _BM), pl.cdiv(N, _BM)),
    in_specs=[
        pl.BlockSpec((_BM, 2 * HID), lambda i, j: (i, 0)),
        pl.BlockSpec((_BM, 2 * HID), lambda i, j: (j, 0)),
    ],
    out_specs=pl.BlockSpec((_BM, _BM), lambda i, j: (i, j)),
    out_shape=jax.ShapeDtypeStruct((N, N), jnp.float32),
    compiler_params=pltpu.CompilerParams(
        dimension_semantics=("parallel", "parallel")),
)


def kernel(x, edge_index, adj, W_fc, b_fc, W1, b1, W2, b2, epsilon, gamma, beta):
    src = edge_index[0]
    dst = edge_index[1]
    pad_src = jnp.full((EPAD - E,), N, dtype=jnp.int32)
    pad_dst = jnp.zeros((EPAD - E,), dtype=jnp.int32)
    e0 = NS * NCHUNK0 * CHUNK

    def _layout(idx, pad):
        flat = jnp.concatenate([idx, pad])
        p0 = flat[:e0].reshape(NS, NCHUNK0, CHUNK)
        p1 = flat[e0:].reshape(NS, NCHUNK1, CHUNK)
        p1 = jnp.pad(p1, ((0, 0), (0, NCHUNK0 - NCHUNK1), (0, 0)))
        return jnp.concatenate([p0, p1], axis=0)

    src_r = _layout(src, pad_src)
    dst_r = _layout(dst, pad_dst)
    zrow = jnp.zeros((1, HID), dtype=jnp.float32)
    zeros_nh = jnp.zeros((NPAD, HID), dtype=jnp.float32)

    seg_sum = _make_seg_sum()
    table1 = jnp.concatenate([x, zrow], axis=0)
    p1 = seg_sum(table1, src_r, dst_r, zeros_nh)
    h2a = _mm_relu(p1, W1, b1.reshape(1, HID))
    table2 = jnp.concatenate([h2a, zrow], axis=0)
    p2 = seg_sum(table2, src_r, dst_r, zeros_nh)
    g, hn = _combine(p2, x, W2, b2.reshape(1, HID), W_fc, b_fc.reshape(1, HID),
                     epsilon.reshape(N, 1), gamma.reshape(1, HID),
                     beta.reshape(1, HID))
    ret = _gram(g, g)
    return (ret, hn)
